# baseline (speedup 1.0000x reference)
import functools

import jax
import jax.numpy as jnp
from jax.experimental import pallas as pl
from jax.experimental.pallas import tpu as pltpu

N = 4096
C = 64
KP1 = 3
BR = 256
NBUF = 4


def _body(x_ref, w_ref, b_ref, lr_hbm, li_hbm, out_ref,
          lrb, lib, rts, rbs, slr, sli):
    def copy(c, slot):
        r, i = divmod(c, KP1)
        return (
            pltpu.make_async_copy(
                lr_hbm.at[i, pl.ds(r * BR, BR), :], lrb.at[slot], slr.at[slot]),
            pltpu.make_async_copy(
                li_hbm.at[i, pl.ds(r * BR, BR), :], lib.at[slot], sli.at[slot]),
        )

    for c in range(NBUF):
        ca, cb = copy(c, c)
        ca.start()
        cb.start()

    xr = x_ref[0]
    xi = x_ref[1]
    for i in range(KP1):
        w = w_ref[i]
        p = jnp.dot(xr, w, preferred_element_type=jnp.float32)
        q = jnp.dot(xi, w, preferred_element_type=jnp.float32)
        rts[i] = jnp.concatenate([p, q], axis=1).astype(jnp.bfloat16)
        rbs[i] = jnp.concatenate([-q, p], axis=1).astype(jnp.bfloat16)

    bb = jnp.concatenate([b_ref[...], b_ref[...]], axis=1)

    nc = (N // BR) * KP1
    for r in range(N // BR):
        acc = None
        for i in range(KP1):
            c = r * KP1 + i
            slot = c % NBUF
            ca, cb = copy(c, slot)
            ca.wait()
            cb.wait()
            a = lrb[slot].astype(jnp.bfloat16)
            b2 = lib[slot].astype(jnp.bfloat16)
            part = jnp.dot(a, rts[i], preferred_element_type=jnp.float32)
            part += jnp.dot(b2, rbs[i], preferred_element_type=jnp.float32)
            acc = part if acc is None else acc + part
            if c + NBUF < nc:
                na, nb = copy(c + NBUF, slot)
                na.start()
                nb.start()
        out_ref[pl.ds(r * BR, BR), :] = acc + bb


@functools.partial(jax.jit, static_argnames=("interpret",))
def _cheb_conv_manual(data, L_real, L_imag, weight, bias, interpret=False):
    out = pl.pallas_call(
        _body,
        in_specs=[
            pl.BlockSpec(memory_space=pltpu.MemorySpace.VMEM),
            pl.BlockSpec(memory_space=pltpu.MemorySpace.VMEM),
            pl.BlockSpec(memory_space=pltpu.MemorySpace.VMEM),
            pl.BlockSpec(memory_space=pl.ANY),
            pl.BlockSpec(memory_space=pl.ANY),
        ],
        out_specs=pl.BlockSpec(memory_space=pltpu.MemorySpace.VMEM),
        out_shape=jax.ShapeDtypeStruct((N, 2 * C), jnp.float32),
        scratch_shapes=[
            pltpu.VMEM((NBUF, BR, N), jnp.float32),
            pltpu.VMEM((NBUF, BR, N), jnp.float32),
            pltpu.VMEM((KP1, N, 2 * C), jnp.bfloat16),
            pltpu.VMEM((KP1, N, 2 * C), jnp.bfloat16),
            pltpu.SemaphoreType.DMA((NBUF,)),
            pltpu.SemaphoreType.DMA((NBUF,)),
        ],
        compiler_params=pltpu.CompilerParams(
            vmem_limit_bytes=63 * 1024 * 1024,
        ),
        interpret=interpret,
    )(data, weight, bias, L_real, L_imag)
    return out[:, :C], out[:, C:]


def kernel(data, L_real, L_imag, weight, bias):
    return _cheb_conv_manual(data, L_real, L_imag, weight, bias)


# speedup vs baseline: 1.0178x; 1.0178x over previous
import functools

import jax
import jax.numpy as jnp
from jax.experimental import pallas as pl
from jax.experimental.pallas import tpu as pltpu

N = 4096
C = 64
KP1 = 3
BR = 128
NBUF = 6


def _body(x_ref, w_ref, b_ref, lr_hbm, li_hbm, out_ref,
          lrb, lib, rts, rbs, slr, sli):
    def copy(c, slot):
        r, i = divmod(c, KP1)
        return (
            pltpu.make_async_copy(
                lr_hbm.at[i, pl.ds(r * BR, BR), :], lrb.at[slot], slr.at[slot]),
            pltpu.make_async_copy(
                li_hbm.at[i, pl.ds(r * BR, BR), :], lib.at[slot], sli.at[slot]),
        )

    for c in range(NBUF):
        ca, cb = copy(c, c)
        ca.start()
        cb.start()

    xr = x_ref[0]
    xi = x_ref[1]
    for i in range(KP1):
        w = w_ref[i]
        p = jnp.dot(xr, w, preferred_element_type=jnp.float32)
        q = jnp.dot(xi, w, preferred_element_type=jnp.float32)
        rts[i] = jnp.concatenate([p, q], axis=1).astype(jnp.bfloat16)
        rbs[i] = jnp.concatenate([-q, p], axis=1).astype(jnp.bfloat16)

    bb = jnp.concatenate([b_ref[...], b_ref[...]], axis=1)

    nc = (N // BR) * KP1
    for r in range(N // BR):
        acc = None
        for i in range(KP1):
            c = r * KP1 + i
            slot = c % NBUF
            ca, cb = copy(c, slot)
            ca.wait()
            cb.wait()
            a = lrb[slot].astype(jnp.bfloat16)
            b2 = lib[slot].astype(jnp.bfloat16)
            part = jnp.dot(a, rts[i], preferred_element_type=jnp.float32)
            part += jnp.dot(b2, rbs[i], preferred_element_type=jnp.float32)
            acc = part if acc is None else acc + part
            if c + NBUF < nc:
                na, nb = copy(c + NBUF, slot)
                na.start()
                nb.start()
        out_ref[pl.ds(r * BR, BR), :] = acc + bb


@functools.partial(jax.jit, static_argnames=("interpret",))
def _cheb_conv_manual(data, L_real, L_imag, weight, bias, interpret=False):
    out = pl.pallas_call(
        _body,
        in_specs=[
            pl.BlockSpec(memory_space=pltpu.MemorySpace.VMEM),
            pl.BlockSpec(memory_space=pltpu.MemorySpace.VMEM),
            pl.BlockSpec(memory_space=pltpu.MemorySpace.VMEM),
            pl.BlockSpec(memory_space=pl.ANY),
            pl.BlockSpec(memory_space=pl.ANY),
        ],
        out_specs=pl.BlockSpec(memory_space=pltpu.MemorySpace.VMEM),
        out_shape=jax.ShapeDtypeStruct((N, 2 * C), jnp.float32),
        scratch_shapes=[
            pltpu.VMEM((NBUF, BR, N), jnp.float32),
            pltpu.VMEM((NBUF, BR, N), jnp.float32),
            pltpu.VMEM((KP1, N, 2 * C), jnp.bfloat16),
            pltpu.VMEM((KP1, N, 2 * C), jnp.bfloat16),
            pltpu.SemaphoreType.DMA((NBUF,)),
            pltpu.SemaphoreType.DMA((NBUF,)),
        ],
        compiler_params=pltpu.CompilerParams(
            vmem_limit_bytes=63 * 1024 * 1024,
        ),
        interpret=interpret,
    )(data, weight, bias, L_real, L_imag)
    return out[:, :C], out[:, C:]


def kernel(data, L_real, L_imag, weight, bias):
    return _cheb_conv_manual(data, L_real, L_imag, weight, bias)
